# Initial kernel scaffold; baseline (speedup 1.0000x reference)
import jax
import jax.numpy as jnp
from jax.experimental import pallas as pl


def kernel(x, W, b):
    raise NotImplementedError("write your pallas kernel here")


# speedup vs baseline: 35.1698x; 35.1698x over previous
import functools
import jax
import jax.numpy as jnp
from jax import lax
from jax.experimental import pallas as pl
from jax.experimental.pallas import tpu as pltpu
from jax.experimental.pallas import tpu_sc as plsc

E = 16
CAP = 2048
BLK = 512
N_TOK = 8192
NSUB = 16
TILE_TOK = N_TOK // NSUB
ROWS_T = TILE_TOK * E // 128
TPR = 128 // E


def _router_body(x_ref, w_ref, b_ref, e1_ref, e2_ref, g0_ref, g1_ref):
    x = x_ref[...]
    w = w_ref[...]
    b = b_ref[...][0:1, :]
    logits = jax.lax.dot(x, w, preferred_element_type=jnp.float32) + b

    lane = jax.lax.broadcasted_iota(jnp.int32, (BLK, E), 1)
    i1 = jnp.argmax(logits, axis=1)[:, None]
    is1 = lane == i1
    masked = jnp.where(is1, jnp.float32(-1e30), logits)
    i2 = jnp.argmax(masked, axis=1)[:, None]

    m1 = jnp.max(logits, axis=1, keepdims=True)
    m2 = jnp.max(masked, axis=1, keepdims=True)
    e_ = jnp.exp(m2 - m1)
    g0 = 1.0 / (1.0 + e_)
    g1 = 1.0 - g0

    e1_ref[...] = jnp.broadcast_to(i1, (BLK, E))
    e2_ref[...] = jnp.broadcast_to(i2, (BLK, E))
    g0_ref[...] = jnp.broadcast_to(g0, (BLK, E))
    g1_ref[...] = jnp.broadcast_to(g1, (BLK, E))


def _sc_body(e1_hbm, e2_hbm, k0_hbm, k1_hbm,
             e1_v, e2_v, hbuf, hall, kv_buf, hshared):
    c = lax.axis_index("c")
    s = lax.axis_index("s")
    iota = lax.iota(jnp.int32, E)
    base_row = s * ROWS_T

    pltpu.sync_copy(e1_hbm.at[pl.ds(base_row, ROWS_T), :], e1_v)
    pltpu.sync_copy(e2_hbm.at[pl.ds(base_row, ROWS_T), :], e2_v)

    def hist_step(r, carry):
        h0, h1 = carry
        for j in range(TPR):
            r1 = e1_v[r, pl.ds(j * E, E)]
            r2 = e2_v[r, pl.ds(j * E, E)]
            h0 = h0 + jnp.where(r1 == iota, 1, 0).astype(jnp.int32)
            h1 = h1 + jnp.where(r2 == iota, 1, 0).astype(jnp.int32)
        return h0, h1

    z = jnp.zeros((E,), jnp.int32)
    h0, h1 = lax.fori_loop(0, ROWS_T, hist_step, (z, z))
    hbuf[0] = h0
    hbuf[1] = h1
    pltpu.sync_copy(hbuf.at[0], hshared.at[s])
    pltpu.sync_copy(hbuf.at[1], hshared.at[NSUB + s])
    plsc.subcore_barrier()
    pltpu.sync_copy(hshared, hall)

    def base_step(sp, carry):
        b0, b1, t0 = carry
        r0 = hall[sp]
        r1 = hall[NSUB + sp]
        use = sp < s
        b0 = b0 + jnp.where(use, r0, 0)
        b1 = b1 + jnp.where(use, r1, 0)
        t0 = t0 + r0
        return b0, b1, t0

    b0, b1, tot0 = lax.fori_loop(0, NSUB, base_step, (z, z, z))

    def make_phase(ev, start_cnt, kout):
        def row_step(r, cnt):
            for j in range(TPR):
                row = ev[r, pl.ds(j * E, E)]
                ohb = row == iota
                kb = ohb & (cnt < CAP)
                kv_buf[r, pl.ds(j * E, E)] = jnp.where(
                    kb, jnp.float32(1.0), jnp.float32(0.0))
                cnt = cnt + jnp.where(ohb, 1, 0).astype(jnp.int32)
            return cnt

        lax.fori_loop(0, ROWS_T, row_step, start_cnt)
        pltpu.sync_copy(kv_buf, kout.at[pl.ds(base_row, ROWS_T), :])

    @pl.when(c == 0)
    def _k0():
        make_phase(e1_v, b0, k0_hbm)

    @pl.when(c == 1)
    def _k1():
        make_phase(e2_v, tot0 + b1, k1_hbm)


def _scale_body(x_ref, g0_ref, g1_ref, k0_ref, k1_ref, o_ref):
    scale = jnp.sum(g0_ref[...] * k0_ref[...] + g1_ref[...] * k1_ref[...],
                    axis=1, keepdims=True)
    o_ref[...] = x_ref[...] * scale


def kernel(x, W, b):
    N, D = x.shape
    nblk = N // BLK
    b8 = jnp.broadcast_to(b.reshape(1, E), (8, E))

    e1b, e2b, g0b, g1b = pl.pallas_call(
        _router_body,
        grid=(nblk,),
        in_specs=[
            pl.BlockSpec((BLK, D), lambda g: (g, 0)),
            pl.BlockSpec((D, E), lambda g: (0, 0)),
            pl.BlockSpec((8, E), lambda g: (0, 0)),
        ],
        out_specs=[
            pl.BlockSpec((BLK, E), lambda g: (g, 0)),
            pl.BlockSpec((BLK, E), lambda g: (g, 0)),
            pl.BlockSpec((BLK, E), lambda g: (g, 0)),
            pl.BlockSpec((BLK, E), lambda g: (g, 0)),
        ],
        out_shape=[
            jax.ShapeDtypeStruct((N, E), jnp.int32),
            jax.ShapeDtypeStruct((N, E), jnp.int32),
            jax.ShapeDtypeStruct((N, E), jnp.float32),
            jax.ShapeDtypeStruct((N, E), jnp.float32),
        ],
    )(x, W, b8)

    nrow = N * E // 128
    e1p = e1b.reshape(nrow, 128)
    e2p = e2b.reshape(nrow, 128)

    mesh = plsc.VectorSubcoreMesh(core_axis_name="c", subcore_axis_name="s")
    sc = functools.partial(
        pl.kernel,
        mesh=mesh,
        out_type=[
            jax.ShapeDtypeStruct((nrow, 128), jnp.float32),
            jax.ShapeDtypeStruct((nrow, 128), jnp.float32),
        ],
        scratch_types=[
            pltpu.VMEM((ROWS_T, 128), jnp.int32),
            pltpu.VMEM((ROWS_T, 128), jnp.int32),
            pltpu.VMEM((2, E), jnp.int32),
            pltpu.VMEM((2 * NSUB, E), jnp.int32),
            pltpu.VMEM((ROWS_T, 128), jnp.float32),
            pltpu.VMEM_SHARED((2 * NSUB, E), jnp.int32),
        ],
    )(_sc_body)
    k0p, k1p = sc(e1p, e2p)
    k0c = k0p.reshape(N, E)
    k1c = k1p.reshape(N, E)

    out = pl.pallas_call(
        _scale_body,
        grid=(nblk,),
        in_specs=[
            pl.BlockSpec((BLK, D), lambda g: (g, 0)),
            pl.BlockSpec((BLK, E), lambda g: (g, 0)),
            pl.BlockSpec((BLK, E), lambda g: (g, 0)),
            pl.BlockSpec((BLK, E), lambda g: (g, 0)),
            pl.BlockSpec((BLK, E), lambda g: (g, 0)),
        ],
        out_specs=pl.BlockSpec((BLK, D), lambda g: (g, 0)),
        out_shape=jax.ShapeDtypeStruct((N, D), jnp.float32),
    )(x, g0b, g1b, k0c, k1c)
    return out


# speedup vs baseline: 38.4712x; 1.0939x over previous
import functools
import jax
import jax.numpy as jnp
from jax import lax
from jax.experimental import pallas as pl
from jax.experimental.pallas import tpu as pltpu
from jax.experimental.pallas import tpu_sc as plsc

E = 16
CAP = 2048
BLK = 512
N_TOK = 8192
NSUB = 16
TILE_TOK = N_TOK // NSUB
TPR = 128 // E
ROWS_T = TILE_TOK // TPR
SUB_ROWS = ROWS_T // 2


def _router_body(x_ref, w_ref, b_ref, i12_ref, g0_ref, g1_ref):
    x = x_ref[...]
    w = w_ref[...]
    b = b_ref[...][0:1, :]
    logits = jax.lax.dot(x, w, preferred_element_type=jnp.float32) + b

    lane = jax.lax.broadcasted_iota(jnp.int32, (BLK, E), 1)
    i1 = jnp.argmax(logits, axis=1)[:, None]
    is1 = lane == i1
    masked = jnp.where(is1, jnp.float32(-1e30), logits)
    i2 = jnp.argmax(masked, axis=1)[:, None]

    m1 = jnp.max(logits, axis=1, keepdims=True)
    m2 = jnp.max(masked, axis=1, keepdims=True)
    e_ = jnp.exp(m2 - m1)
    g0 = 1.0 / (1.0 + e_)
    g1 = 1.0 - g0

    colf = (i1 + E * i2).astype(jnp.float32)
    ri = jax.lax.broadcasted_iota(jnp.int32, (BLK // TPR, BLK), 0)
    ti = jax.lax.broadcasted_iota(jnp.int32, (BLK // TPR, BLK), 1)
    fgrp = (ti // TPR == ri).astype(jnp.float32)
    tm = jax.lax.broadcasted_iota(jnp.int32, (BLK, 128), 0) % TPR
    lg = jax.lax.broadcasted_iota(jnp.int32, (BLK, 128), 1) // E
    m = (lg == tm).astype(jnp.float32)
    packed = jax.lax.dot(fgrp, colf * m,
                         preferred_element_type=jnp.float32)
    i12_ref[...] = packed.astype(jnp.int32)
    g0_ref[...] = jnp.broadcast_to(g0, (BLK, E))
    g1_ref[...] = jnp.broadcast_to(g1, (BLK, E))


def _sc_body(i12_hbm, k01_hbm, e_v, hbuf, hall, kv_buf, hshared):
    c = lax.axis_index("c")
    s = lax.axis_index("s")
    iota = lax.iota(jnp.int32, E)
    base_row = s * ROWS_T

    pltpu.sync_copy(i12_hbm.at[pl.ds(base_row, ROWS_T), :], e_v)

    def hist_step(r, carry):
        h0, h1 = carry
        for j in range(TPR):
            v = e_v[r, pl.ds(j * E, E)]
            v1 = v & (E - 1)
            v2 = lax.shift_right_logical(v, 4)
            h0 = h0 + jnp.where(v1 == iota, 1, 0).astype(jnp.int32)
            h1 = h1 + jnp.where(v2 == iota, 1, 0).astype(jnp.int32)
        return h0, h1

    z = jnp.zeros((E,), jnp.int32)
    h0a, h1a = lax.fori_loop(0, SUB_ROWS, hist_step, (z, z))
    h0b, h1b = lax.fori_loop(SUB_ROWS, ROWS_T, hist_step, (z, z))
    hbuf[0] = h0a
    hbuf[1] = h0b
    hbuf[2] = h1a
    hbuf[3] = h1b
    pltpu.sync_copy(hbuf.at[0], hshared.at[2 * s])
    pltpu.sync_copy(hbuf.at[1], hshared.at[2 * s + 1])
    pltpu.sync_copy(hbuf.at[2], hshared.at[2 * NSUB + 2 * s])
    pltpu.sync_copy(hbuf.at[3], hshared.at[2 * NSUB + 2 * s + 1])
    plsc.subcore_barrier()
    pltpu.sync_copy(hshared, hall)

    q = 2 * s + c

    def base_step(sp, carry):
        b0, b1, t0 = carry
        r0 = hall[sp]
        r1 = hall[2 * NSUB + sp]
        use = sp < q
        b0 = b0 + jnp.where(use, r0, 0)
        b1 = b1 + jnp.where(use, r1, 0)
        t0 = t0 + r0
        return b0, b1, t0

    b0, b1, tot0 = lax.fori_loop(0, 2 * NSUB, base_step, (z, z, z))

    row_lo = c * SUB_ROWS

    def tok_step(r, carry):
        cnt0, cnt1 = carry
        for j in range(TPR):
            v = e_v[row_lo + r, pl.ds(j * E, E)]
            oh0 = (v & (E - 1)) == iota
            oh1 = lax.shift_right_logical(v, 4) == iota
            kb0 = oh0 & (cnt0 < CAP)
            kb1 = oh1 & (cnt1 < CAP)
            kv_buf[r, pl.ds(j * E, E)] = (
                jnp.where(kb0, jnp.float32(1.0), jnp.float32(0.0))
                + jnp.where(kb1, jnp.float32(2.0), jnp.float32(0.0)))
            cnt0 = cnt0 + jnp.where(oh0, 1, 0).astype(jnp.int32)
            cnt1 = cnt1 + jnp.where(oh1, 1, 0).astype(jnp.int32)
        return cnt0, cnt1

    lax.fori_loop(0, SUB_ROWS, tok_step, (b0, tot0 + b1))
    pltpu.sync_copy(
        kv_buf, k01_hbm.at[pl.ds(base_row + row_lo, SUB_ROWS), :])


def _scale_body(x_ref, g0_ref, g1_ref, k01_ref, o_ref):
    k01 = k01_ref[...]
    k0p = (k01 == 1.0).astype(jnp.float32)
    k1p = (k01 == 2.0).astype(jnp.float32)
    ti = jax.lax.broadcasted_iota(jnp.int32, (BLK, BLK // TPR), 0)
    ri = jax.lax.broadcasted_iota(jnp.int32, (BLK, BLK // TPR), 1)
    ft = (ti // TPR == ri).astype(jnp.float32)
    tm = jax.lax.broadcasted_iota(jnp.int32, (BLK, 128), 0) % TPR
    lg = jax.lax.broadcasted_iota(jnp.int32, (BLK, 128), 1) // E
    m = (lg == tm).astype(jnp.float32)
    u0 = jax.lax.dot(ft, k0p, preferred_element_type=jnp.float32)
    u1 = jax.lax.dot(ft, k1p, preferred_element_type=jnp.float32)
    keep0 = jnp.sum(m * u0, axis=1, keepdims=True)
    keep1 = jnp.sum(m * u1, axis=1, keepdims=True)
    scale = g0_ref[...][:, 0:1] * keep0 + g1_ref[...][:, 0:1] * keep1
    o_ref[...] = x_ref[...] * scale


def kernel(x, W, b):
    N, D = x.shape
    nblk = N // BLK
    nrow = N // TPR
    b8 = jnp.broadcast_to(b.reshape(1, E), (8, E))

    i12p, g0b, g1b = pl.pallas_call(
        _router_body,
        grid=(nblk,),
        in_specs=[
            pl.BlockSpec((BLK, D), lambda g: (g, 0)),
            pl.BlockSpec((D, E), lambda g: (0, 0)),
            pl.BlockSpec((8, E), lambda g: (0, 0)),
        ],
        out_specs=[
            pl.BlockSpec((BLK // TPR, 128), lambda g: (g, 0)),
            pl.BlockSpec((BLK, E), lambda g: (g, 0)),
            pl.BlockSpec((BLK, E), lambda g: (g, 0)),
        ],
        out_shape=[
            jax.ShapeDtypeStruct((nrow, 128), jnp.int32),
            jax.ShapeDtypeStruct((N, E), jnp.float32),
            jax.ShapeDtypeStruct((N, E), jnp.float32),
        ],
    )(x, W, b8)

    mesh = plsc.VectorSubcoreMesh(core_axis_name="c", subcore_axis_name="s")
    sc = functools.partial(
        pl.kernel,
        mesh=mesh,
        out_type=jax.ShapeDtypeStruct((nrow, 128), jnp.float32),
        scratch_types=[
            pltpu.VMEM((ROWS_T, 128), jnp.int32),
            pltpu.VMEM((4, E), jnp.int32),
            pltpu.VMEM((4 * NSUB, E), jnp.int32),
            pltpu.VMEM((SUB_ROWS, 128), jnp.float32),
            pltpu.VMEM_SHARED((4 * NSUB, E), jnp.int32),
        ],
    )(_sc_body)
    k01p = sc(i12p)

    out = pl.pallas_call(
        _scale_body,
        grid=(nblk,),
        in_specs=[
            pl.BlockSpec((BLK, D), lambda g: (g, 0)),
            pl.BlockSpec((BLK, E), lambda g: (g, 0)),
            pl.BlockSpec((BLK, E), lambda g: (g, 0)),
            pl.BlockSpec((BLK // TPR, 128), lambda g: (g, 0)),
        ],
        out_specs=pl.BlockSpec((BLK, D), lambda g: (g, 0)),
        out_shape=jax.ShapeDtypeStruct((N, D), jnp.float32),
    )(x, g0b, g1b, k01p)
    return out


# speedup vs baseline: 39.0343x; 1.0146x over previous
import functools
import jax
import jax.numpy as jnp
from jax import lax
from jax.experimental import pallas as pl
from jax.experimental.pallas import tpu as pltpu
from jax.experimental.pallas import tpu_sc as plsc

E = 16
CAP = 2048
BLK = 512
N_TOK = 8192
NSUB = 16
TILE_TOK = N_TOK // NSUB
TPR = 128 // E
ROWS_T = TILE_TOK // TPR
SUB_ROWS = ROWS_T // 2


def _router_body(x_ref, w_ref, b_ref, i12_ref, g0_ref):
    x = x_ref[...]
    w = w_ref[...]
    b = b_ref[...][0:1, :]
    logits = jax.lax.dot(x, w, preferred_element_type=jnp.float32) + b

    lane = jax.lax.broadcasted_iota(jnp.int32, (BLK, E), 1)
    i1 = jnp.argmax(logits, axis=1)[:, None]
    is1 = lane == i1
    masked = jnp.where(is1, jnp.float32(-1e30), logits)
    i2 = jnp.argmax(masked, axis=1)[:, None]

    m1 = jnp.max(logits, axis=1, keepdims=True)
    m2 = jnp.max(masked, axis=1, keepdims=True)
    e_ = jnp.exp(m2 - m1)
    g0 = 1.0 / (1.0 + e_)

    colf = (i1 + E * i2).astype(jnp.float32)
    ri = jax.lax.broadcasted_iota(jnp.int32, (BLK // TPR, BLK), 0)
    ti = jax.lax.broadcasted_iota(jnp.int32, (BLK // TPR, BLK), 1)
    fgrp = (ti // TPR == ri).astype(jnp.float32)
    tm = jax.lax.broadcasted_iota(jnp.int32, (BLK, 128), 0) % TPR
    lg = jax.lax.broadcasted_iota(jnp.int32, (BLK, 128), 1) // E
    m = (lg == tm).astype(jnp.float32)
    packed = jax.lax.dot(fgrp, colf * m,
                         preferred_element_type=jnp.float32)
    i12_ref[...] = packed.astype(jnp.int32)
    g0_ref[...] = jnp.broadcast_to(g0, (BLK, E))


def _sc_body(i12_hbm, k01_hbm, e_v, hbuf, hall, kv_buf, hshared):
    c = lax.axis_index("c")
    s = lax.axis_index("s")
    iota = lax.iota(jnp.int32, E)
    base_row = s * ROWS_T

    pltpu.sync_copy(i12_hbm.at[pl.ds(base_row, ROWS_T), :], e_v)

    def hist_step(r, carry):
        h0, h1 = carry
        for j in range(TPR):
            v = e_v[r, pl.ds(j * E, E)]
            v1 = v & (E - 1)
            v2 = lax.shift_right_logical(v, 4)
            h0 = h0 + jnp.where(v1 == iota, 1, 0).astype(jnp.int32)
            h1 = h1 + jnp.where(v2 == iota, 1, 0).astype(jnp.int32)
        return h0, h1

    z = jnp.zeros((E,), jnp.int32)
    h0a, h1a = lax.fori_loop(0, SUB_ROWS, hist_step, (z, z))
    h0b, h1b = lax.fori_loop(SUB_ROWS, ROWS_T, hist_step, (z, z))
    hbuf[0] = h0a
    hbuf[1] = h0b
    hbuf[2] = h1a
    hbuf[3] = h1b
    pltpu.sync_copy(hbuf.at[0], hshared.at[2 * s])
    pltpu.sync_copy(hbuf.at[1], hshared.at[2 * s + 1])
    pltpu.sync_copy(hbuf.at[2], hshared.at[2 * NSUB + 2 * s])
    pltpu.sync_copy(hbuf.at[3], hshared.at[2 * NSUB + 2 * s + 1])
    plsc.subcore_barrier()
    pltpu.sync_copy(hshared, hall)

    q = 2 * s + c

    def base_step(sp, carry):
        b0, b1, t0 = carry
        r0 = hall[sp]
        r1 = hall[2 * NSUB + sp]
        use = sp < q
        b0 = b0 + jnp.where(use, r0, 0)
        b1 = b1 + jnp.where(use, r1, 0)
        t0 = t0 + r0
        return b0, b1, t0

    b0, b1, tot0 = lax.fori_loop(0, 2 * NSUB, base_step, (z, z, z))

    row_lo = c * SUB_ROWS

    def tok_step(r, carry):
        cnt0, cnt1 = carry
        for j in range(TPR):
            v = e_v[row_lo + r, pl.ds(j * E, E)]
            oh0 = (v & (E - 1)) == iota
            oh1 = lax.shift_right_logical(v, 4) == iota
            kb0 = oh0 & (cnt0 < CAP)
            kb1 = oh1 & (cnt1 < CAP)
            kv_buf[r, pl.ds(j * E, E)] = (
                jnp.where(kb0, jnp.float32(1.0), jnp.float32(0.0))
                + jnp.where(kb1, jnp.float32(2.0), jnp.float32(0.0)))
            cnt0 = cnt0 + jnp.where(oh0, 1, 0).astype(jnp.int32)
            cnt1 = cnt1 + jnp.where(oh1, 1, 0).astype(jnp.int32)
        return cnt0, cnt1

    lax.fori_loop(0, SUB_ROWS, tok_step, (b0, tot0 + b1))
    pltpu.sync_copy(
        kv_buf, k01_hbm.at[pl.ds(base_row + row_lo, SUB_ROWS), :])


def _scale_body(x_ref, g0_ref, k01_ref, o_ref):
    k01 = k01_ref[...]
    k0p = (k01 == 1.0).astype(jnp.float32)
    k1p = (k01 == 2.0).astype(jnp.float32)
    ti = jax.lax.broadcasted_iota(jnp.int32, (BLK, BLK // TPR), 0)
    ri = jax.lax.broadcasted_iota(jnp.int32, (BLK, BLK // TPR), 1)
    ft = (ti // TPR == ri).astype(jnp.float32)
    tm = jax.lax.broadcasted_iota(jnp.int32, (BLK, 128), 0) % TPR
    lg = jax.lax.broadcasted_iota(jnp.int32, (BLK, 128), 1) // E
    m = (lg == tm).astype(jnp.float32)
    u0 = jax.lax.dot(ft, k0p, preferred_element_type=jnp.float32)
    u1 = jax.lax.dot(ft, k1p, preferred_element_type=jnp.float32)
    keep0 = jnp.sum(m * u0, axis=1, keepdims=True)
    keep1 = jnp.sum(m * u1, axis=1, keepdims=True)
    g0 = g0_ref[...][:, 0:1]
    scale = g0 * keep0 + (1.0 - g0) * keep1
    o_ref[...] = x_ref[...] * scale


def kernel(x, W, b):
    N, D = x.shape
    nblk = N // BLK
    nrow = N // TPR
    b8 = jnp.broadcast_to(b.reshape(1, E), (8, E))

    i12p, g0b = pl.pallas_call(
        _router_body,
        grid=(nblk,),
        in_specs=[
            pl.BlockSpec((BLK, D), lambda g: (g, 0)),
            pl.BlockSpec((D, E), lambda g: (0, 0)),
            pl.BlockSpec((8, E), lambda g: (0, 0)),
        ],
        out_specs=[
            pl.BlockSpec((BLK // TPR, 128), lambda g: (g, 0)),
            pl.BlockSpec((BLK, E), lambda g: (g, 0)),
        ],
        out_shape=[
            jax.ShapeDtypeStruct((nrow, 128), jnp.int32),
            jax.ShapeDtypeStruct((N, E), jnp.float32),
        ],
    )(x, W, b8)

    mesh = plsc.VectorSubcoreMesh(core_axis_name="c", subcore_axis_name="s")
    sc = functools.partial(
        pl.kernel,
        mesh=mesh,
        out_type=jax.ShapeDtypeStruct((nrow, 128), jnp.float32),
        scratch_types=[
            pltpu.VMEM((ROWS_T, 128), jnp.int32),
            pltpu.VMEM((4, E), jnp.int32),
            pltpu.VMEM((4 * NSUB, E), jnp.int32),
            pltpu.VMEM((SUB_ROWS, 128), jnp.float32),
            pltpu.VMEM_SHARED((4 * NSUB, E), jnp.int32),
        ],
    )(_sc_body)
    k01p = sc(i12p)

    out = pl.pallas_call(
        _scale_body,
        grid=(nblk,),
        in_specs=[
            pl.BlockSpec((BLK, D), lambda g: (g, 0)),
            pl.BlockSpec((BLK, E), lambda g: (g, 0)),
            pl.BlockSpec((BLK // TPR, 128), lambda g: (g, 0)),
        ],
        out_specs=pl.BlockSpec((BLK, D), lambda g: (g, 0)),
        out_shape=jax.ShapeDtypeStruct((N, D), jnp.float32),
    )(x, g0b, k01p)
    return out


# speedup vs baseline: 39.1605x; 1.0032x over previous
import functools
import jax
import jax.numpy as jnp
from jax import lax
from jax.experimental import pallas as pl
from jax.experimental.pallas import tpu as pltpu
from jax.experimental.pallas import tpu_sc as plsc

E = 16
CAP = 2048
BLK = 512
BLK_R = 1024
N_TOK = 8192
NSUB = 16
TILE_TOK = N_TOK // NSUB
TPR = 128 // E
ROWS_T = TILE_TOK // TPR
SUB_ROWS = ROWS_T // 2


def _router_body(x_ref, w_ref, b_ref, i12_ref, g0_ref):
    x = x_ref[...]
    w = w_ref[...]
    b = b_ref[...][0:1, :]
    logits = jax.lax.dot(x, w, preferred_element_type=jnp.float32) + b

    lane = jax.lax.broadcasted_iota(jnp.int32, (BLK_R, E), 1)
    i1 = jnp.argmax(logits, axis=1)[:, None]
    is1 = lane == i1
    masked = jnp.where(is1, jnp.float32(-1e30), logits)
    i2 = jnp.argmax(masked, axis=1)[:, None]

    m1 = jnp.max(logits, axis=1, keepdims=True)
    m2 = jnp.max(masked, axis=1, keepdims=True)
    e_ = jnp.exp(m2 - m1)
    g0 = 1.0 / (1.0 + e_)

    colf = (i1 + E * i2).astype(jnp.float32)
    ri = jax.lax.broadcasted_iota(jnp.int32, (BLK_R // TPR, BLK_R), 0)
    ti = jax.lax.broadcasted_iota(jnp.int32, (BLK_R // TPR, BLK_R), 1)
    fgrp = (ti // TPR == ri).astype(jnp.float32)
    tm = jax.lax.broadcasted_iota(jnp.int32, (BLK_R, 128), 0) % TPR
    lg = jax.lax.broadcasted_iota(jnp.int32, (BLK_R, 128), 1) // E
    m = (lg == tm).astype(jnp.float32)
    packed = jax.lax.dot(fgrp, colf * m,
                         preferred_element_type=jnp.float32)
    i12_ref[...] = packed.astype(jnp.int32)
    g0_ref[...] = jnp.broadcast_to(g0, (BLK_R, E))


def _sc_body(i12_hbm, k01_hbm, e_v, hbuf, hall, kv_buf, hshared):
    c = lax.axis_index("c")
    s = lax.axis_index("s")
    iota = lax.iota(jnp.int32, E)
    base_row = s * ROWS_T

    pltpu.sync_copy(i12_hbm.at[pl.ds(base_row, ROWS_T), :], e_v)

    def hist_step(r, carry):
        h0, h1 = carry
        for j in range(TPR):
            v = e_v[r, pl.ds(j * E, E)]
            v1 = v & (E - 1)
            v2 = lax.shift_right_logical(v, 4)
            h0 = h0 + jnp.where(v1 == iota, 1, 0).astype(jnp.int32)
            h1 = h1 + jnp.where(v2 == iota, 1, 0).astype(jnp.int32)
        return h0, h1

    z = jnp.zeros((E,), jnp.int32)
    h0a, h1a = lax.fori_loop(0, SUB_ROWS, hist_step, (z, z))
    h0b, h1b = lax.fori_loop(SUB_ROWS, ROWS_T, hist_step, (z, z))
    hbuf[0] = h0a
    hbuf[1] = h0b
    hbuf[2] = h1a
    hbuf[3] = h1b
    pltpu.sync_copy(hbuf.at[0], hshared.at[2 * s])
    pltpu.sync_copy(hbuf.at[1], hshared.at[2 * s + 1])
    pltpu.sync_copy(hbuf.at[2], hshared.at[2 * NSUB + 2 * s])
    pltpu.sync_copy(hbuf.at[3], hshared.at[2 * NSUB + 2 * s + 1])
    plsc.subcore_barrier()
    pltpu.sync_copy(hshared, hall)

    q = 2 * s + c

    def base_step(sp, carry):
        b0, b1, t0 = carry
        r0 = hall[sp]
        r1 = hall[2 * NSUB + sp]
        use = sp < q
        b0 = b0 + jnp.where(use, r0, 0)
        b1 = b1 + jnp.where(use, r1, 0)
        t0 = t0 + r0
        return b0, b1, t0

    b0, b1, tot0 = lax.fori_loop(0, 2 * NSUB, base_step, (z, z, z))

    row_lo = c * SUB_ROWS

    def tok_step(r, carry):
        cnt0, cnt1 = carry
        for j in range(TPR):
            v = e_v[row_lo + r, pl.ds(j * E, E)]
            oh0 = (v & (E - 1)) == iota
            oh1 = lax.shift_right_logical(v, 4) == iota
            kb0 = oh0 & (cnt0 < CAP)
            kb1 = oh1 & (cnt1 < CAP)
            kv_buf[r, pl.ds(j * E, E)] = (
                jnp.where(kb0, jnp.float32(1.0), jnp.float32(0.0))
                + jnp.where(kb1, jnp.float32(2.0), jnp.float32(0.0)))
            cnt0 = cnt0 + jnp.where(oh0, 1, 0).astype(jnp.int32)
            cnt1 = cnt1 + jnp.where(oh1, 1, 0).astype(jnp.int32)
        return cnt0, cnt1

    lax.fori_loop(0, SUB_ROWS, tok_step, (b0, tot0 + b1))
    pltpu.sync_copy(
        kv_buf, k01_hbm.at[pl.ds(base_row + row_lo, SUB_ROWS), :])


def _scale_body(x_ref, g0_ref, k01_ref, o_ref):
    k01 = k01_ref[...]
    k0p = (k01 == 1.0).astype(jnp.float32)
    k1p = (k01 == 2.0).astype(jnp.float32)
    ti = jax.lax.broadcasted_iota(jnp.int32, (BLK, BLK // TPR), 0)
    ri = jax.lax.broadcasted_iota(jnp.int32, (BLK, BLK // TPR), 1)
    ft = (ti // TPR == ri).astype(jnp.float32)
    tm = jax.lax.broadcasted_iota(jnp.int32, (BLK, 128), 0) % TPR
    lg = jax.lax.broadcasted_iota(jnp.int32, (BLK, 128), 1) // E
    m = (lg == tm).astype(jnp.float32)
    u0 = jax.lax.dot(ft, k0p, preferred_element_type=jnp.float32)
    u1 = jax.lax.dot(ft, k1p, preferred_element_type=jnp.float32)
    keep0 = jnp.sum(m * u0, axis=1, keepdims=True)
    keep1 = jnp.sum(m * u1, axis=1, keepdims=True)
    g0 = g0_ref[...][:, 0:1]
    scale = g0 * keep0 + (1.0 - g0) * keep1
    o_ref[...] = x_ref[...] * scale


def kernel(x, W, b):
    N, D = x.shape
    nblk = N // BLK
    nrow = N // TPR
    b8 = jnp.broadcast_to(b.reshape(1, E), (8, E))

    i12p, g0b = pl.pallas_call(
        _router_body,
        grid=(N // BLK_R,),
        in_specs=[
            pl.BlockSpec((BLK_R, D), lambda g: (g, 0)),
            pl.BlockSpec((D, E), lambda g: (0, 0)),
            pl.BlockSpec((8, E), lambda g: (0, 0)),
        ],
        out_specs=[
            pl.BlockSpec((BLK_R // TPR, 128), lambda g: (g, 0)),
            pl.BlockSpec((BLK_R, E), lambda g: (g, 0)),
        ],
        out_shape=[
            jax.ShapeDtypeStruct((nrow, 128), jnp.int32),
            jax.ShapeDtypeStruct((N, E), jnp.float32),
        ],
    )(x, W, b8)

    mesh = plsc.VectorSubcoreMesh(core_axis_name="c", subcore_axis_name="s")
    sc = functools.partial(
        pl.kernel,
        mesh=mesh,
        out_type=jax.ShapeDtypeStruct((nrow, 128), jnp.float32),
        scratch_types=[
            pltpu.VMEM((ROWS_T, 128), jnp.int32),
            pltpu.VMEM((4, E), jnp.int32),
            pltpu.VMEM((4 * NSUB, E), jnp.int32),
            pltpu.VMEM((SUB_ROWS, 128), jnp.float32),
            pltpu.VMEM_SHARED((4 * NSUB, E), jnp.int32),
        ],
    )(_sc_body)
    k01p = sc(i12p)

    out = pl.pallas_call(
        _scale_body,
        grid=(nblk,),
        in_specs=[
            pl.BlockSpec((BLK, D), lambda g: (g, 0)),
            pl.BlockSpec((BLK, E), lambda g: (g, 0)),
            pl.BlockSpec((BLK // TPR, 128), lambda g: (g, 0)),
        ],
        out_specs=pl.BlockSpec((BLK, D), lambda g: (g, 0)),
        out_shape=jax.ShapeDtypeStruct((N, D), jnp.float32),
    )(x, g0b, k01p)
    return out
